# baseline (speedup 1.0000x reference)
import functools

import jax
import jax.numpy as jnp
from jax import lax
from jax.experimental import pallas as pl
from jax.experimental.pallas import tpu as pltpu
from jax.experimental.pallas import tpu_sc as plsc

N = 4096
L = 16
NC = 2
NS = 16
NW = NC * NS
ROWS_PER_W = N // NW
RB = 4


def _flip_sc_body(x_hbm, out_hbm, in_v, out_v):
    wid = lax.axis_index("s") * NC + lax.axis_index("c")
    base = wid * ROWS_PER_W

    def block(b, carry):
        r0 = base + b * RB
        pltpu.sync_copy(x_hbm.at[pl.ds(r0, RB)], in_v)

        def chunk(c, inner):
            src = (N // L - 1 - c) * L
            for i in range(RB):
                v = in_v[i, pl.ds(src, L)]
                out_v[i, pl.ds(c * L, L)] = lax.rev(v, (0,))
            return inner

        lax.fori_loop(0, N // L, chunk, 0)
        pltpu.sync_copy(out_v, out_hbm.at[pl.ds(r0, RB)])
        return carry

    lax.fori_loop(0, ROWS_PER_W // RB, block, 0)


def kernel(inputs):
    flip = functools.partial(
        pl.kernel,
        mesh=plsc.VectorSubcoreMesh(core_axis_name="c", subcore_axis_name="s"),
        out_type=jax.ShapeDtypeStruct((N, N), jnp.float32),
        scratch_types=[
            pltpu.VMEM((RB, N), jnp.float32),
            pltpu.VMEM((RB, N), jnp.float32),
        ],
    )(_flip_sc_body)
    return flip(inputs)


# speedup vs baseline: 2.2534x; 2.2534x over previous
import functools

import jax
import jax.numpy as jnp
from jax import lax
from jax.experimental import pallas as pl
from jax.experimental.pallas import tpu as pltpu
from jax.experimental.pallas import tpu_sc as plsc

N = 4096
L = 16
NC = 2
NS = 16
NW = NC * NS
ROWS_PER_W = N // NW
RB = 4


def _flip_sc_body(x_hbm, out_hbm, in_v, out_v):
    wid = lax.axis_index("s") * NC + lax.axis_index("c")
    base = wid * ROWS_PER_W

    def block(b, carry):
        r0 = base + b * RB
        pltpu.sync_copy(x_hbm.at[pl.ds(r0, RB)], in_v)

        @plsc.parallel_loop(0, N // L, step=1, unroll=8)
        def chunk(c):
            src = (N // L - 1 - c) * L
            for i in range(RB):
                v = in_v[i, pl.ds(src, L)]
                out_v[i, pl.ds(c * L, L)] = lax.rev(v, (0,))

        pltpu.sync_copy(out_v, out_hbm.at[pl.ds(r0, RB)])
        return carry

    lax.fori_loop(0, ROWS_PER_W // RB, block, 0)


def kernel(inputs):
    flip = functools.partial(
        pl.kernel,
        mesh=plsc.VectorSubcoreMesh(core_axis_name="c", subcore_axis_name="s"),
        out_type=jax.ShapeDtypeStruct((N, N), jnp.float32),
        scratch_types=[
            pltpu.VMEM((RB, N), jnp.float32),
            pltpu.VMEM((RB, N), jnp.float32),
        ],
    )(_flip_sc_body)
    return flip(inputs)


# speedup vs baseline: 3.3373x; 1.4810x over previous
import functools

import jax
import jax.numpy as jnp
from jax import lax
from jax.experimental import pallas as pl
from jax.experimental.pallas import tpu as pltpu
from jax.experimental.pallas import tpu_sc as plsc

N = 4096
L = 16
NC = 2
NS = 16
NW = NC * NS
ROWS_PER_W = N // NW
RB = 4
NBUF = 2
NBLK = ROWS_PER_W // RB


def _flip_sc_body(x_hbm, out_hbm, in_v, out_v, in_s0, in_s1, out_s0, out_s1):
    wid = lax.axis_index("s") * NC + lax.axis_index("c")
    base = wid * ROWS_PER_W
    in_sems = (in_s0, in_s1)
    out_sems = (out_s0, out_s1)

    def in_copy(g, b):
        return pltpu.make_async_copy(
            x_hbm.at[pl.ds(base + g * RB, RB)], in_v.at[b], in_sems[b]
        )

    def out_copy(g, b):
        return pltpu.make_async_copy(
            out_v.at[b], out_hbm.at[pl.ds(base + g * RB, RB)], out_sems[b]
        )

    for b in range(NBUF):
        in_copy(b, b).start()

    for g in range(NBLK):
        b = g % NBUF
        in_copy(g, b).wait()
        if g >= NBUF:
            out_copy(g - NBUF, b).wait()

        @plsc.parallel_loop(0, N // L, step=1, unroll=8)
        def chunk(c):
            src = (N // L - 1 - c) * L
            for i in range(RB):
                v = in_v[b, i, pl.ds(src, L)]
                out_v[b, i, pl.ds(c * L, L)] = lax.rev(v, (0,))

        out_copy(g, b).start()
        if g + NBUF < NBLK:
            in_copy(g + NBUF, b).start()

    for g in range(NBLK - NBUF, NBLK):
        out_copy(g, g % NBUF).wait()


def kernel(inputs):
    flip = functools.partial(
        pl.kernel,
        mesh=plsc.VectorSubcoreMesh(core_axis_name="c", subcore_axis_name="s"),
        out_type=jax.ShapeDtypeStruct((N, N), jnp.float32),
        scratch_types=[
            pltpu.VMEM((NBUF, RB, N), jnp.float32),
            pltpu.VMEM((NBUF, RB, N), jnp.float32),
            pltpu.SemaphoreType.DMA,
            pltpu.SemaphoreType.DMA,
            pltpu.SemaphoreType.DMA,
            pltpu.SemaphoreType.DMA,
        ],
    )(_flip_sc_body)
    return flip(inputs)
